# Initial kernel scaffold; baseline (speedup 1.0000x reference)
import jax
import jax.numpy as jnp
from jax.experimental import pallas as pl


def kernel(feats0, feats1, feats2, pos, nei0, nei1, W0, b0, W1, b1, W2, b2, att_i0, att_i1, Wint, bint, att_inter, Wp1, bp1, Wp2, bp2):
    raise NotImplementedError("write your pallas kernel here")


# speedup vs baseline: 1.6244x; 1.6244x over previous
import functools

import jax
import jax.numpy as jnp
from jax import lax
from jax.experimental import pallas as pl
from jax.experimental.pallas import tpu as pltpu
from jax.experimental.pallas import tpu_sc as plsc

_pc = pl.pallas_call

N0 = 5000
H = 64
INV_TAU = 2.0

NB = 5
BN = N0 // NB


def _elu(x):
    return jnp.where(x > 0, x, jnp.exp(x) - 1.0)


def _leaky(x):
    return jnp.where(x >= 0, x, 0.01 * x)


def _proj_body(x_ref, w_ref, b_ref, o_ref):
    h = _elu(
        jnp.dot(x_ref[...], w_ref[...], preferred_element_type=jnp.float32)
        + b_ref[...])
    o_ref[...] = jnp.concatenate([h, jnp.zeros_like(h)], axis=1)


def _project_h1(feats1, W1, b1):
    n, d = feats1.shape
    blk = 2000
    return _pc(
        _proj_body,
        grid=(n // blk,),
        in_specs=[
            pl.BlockSpec((blk, d), lambda i: (i, 0)),
            pl.BlockSpec((d, H), lambda i: (0, 0)),
            pl.BlockSpec((1, H), lambda i: (0, 0)),
        ],
        out_specs=pl.BlockSpec((blk, 2 * H), lambda i: (i, 0)),
        out_shape=jax.ShapeDtypeStruct((n, 2 * H), jnp.float32),
    )(feats1, W1.T, b1.reshape(1, H))


def _sc_gather(table, idx, n_chunks, chunk):
    B = idx.shape[0]
    D = table.shape[1]
    info = plsc.get_sparse_core_info()
    NC, NS = info.num_cores, info.num_subcores
    per_w = B // (NC * NS)
    mesh = plsc.VectorSubcoreMesh(core_axis_name="c", subcore_axis_name="s")

    @functools.partial(
        pl.kernel,
        mesh=mesh,
        out_type=jax.ShapeDtypeStruct((B, D), jnp.float32),
        scratch_types=[
            pltpu.VMEM((chunk,), jnp.int32),
            pltpu.VMEM((chunk, D), jnp.float32),
            pltpu.SemaphoreType.DMA,
        ],
    )
    def gk(idx_hbm, table_hbm, out_hbm, idx_v, rows_v, sem):
        wid = lax.axis_index("s") * NC + lax.axis_index("c")
        base = wid * per_w
        for c in range(n_chunks):
            off = base + c * chunk
            pltpu.sync_copy(idx_hbm.at[pl.ds(off, chunk)], idx_v)
            pltpu.async_copy(table_hbm.at[idx_v], rows_v, sem).wait()
            pltpu.sync_copy(rows_v, out_hbm.at[pl.ds(off, chunk)])

    return gk(idx, table)


def _attn_body(f0_ref, g1_ref, g2_ref, w0_ref, b0_ref, ar0_ref, an0_ref,
               ar1_ref, an1_ref, w2_ref, b2_ref, wi_ref, bi_ref,
               e0_ref, e1_ref, t0_ref, t1_ref):
    h0 = _elu(jnp.dot(f0_ref[...], w0_ref[...],
                      preferred_element_type=jnp.float32) + b0_ref[...])

    r0 = jnp.sum(h0 * ar0_ref[...], axis=1, keepdims=True)
    cols = [jnp.sum(g1_ref[:, s, :] * an0_ref[...], axis=1, keepdims=True)
            for s in range(20)]
    logit = _leaky(jnp.concatenate(cols, axis=1) + r0)
    m = jnp.max(logit, axis=1, keepdims=True)
    w = jnp.exp(logit - m)
    w = w / jnp.sum(w, axis=1, keepdims=True)
    acc = w[:, 0:1] * g1_ref[:, 0, :]
    for s in range(1, 20):
        acc = acc + w[:, s:s + 1] * g1_ref[:, s, :]
    e0 = _elu(acc[:, 0:H])
    e0_ref[...] = e0

    r1 = jnp.sum(h0 * ar1_ref[...], axis=1, keepdims=True)
    hs = [_elu(jnp.dot(g2_ref[:, s, :], w2_ref[...],
                       preferred_element_type=jnp.float32) + b2_ref[...])
          for s in range(5)]
    cols = [jnp.sum(h * an1_ref[...], axis=1, keepdims=True) for h in hs]
    logit = _leaky(jnp.concatenate(cols, axis=1) + r1)
    m = jnp.max(logit, axis=1, keepdims=True)
    w = jnp.exp(logit - m)
    w = w / jnp.sum(w, axis=1, keepdims=True)
    acc = w[:, 0:1] * hs[0]
    for s in range(1, 5):
        acc = acc + w[:, s:s + 1] * hs[s]
    e1 = _elu(acc)
    e1_ref[...] = e1

    t0_ref[...] = jnp.sum(
        jnp.tanh(jnp.dot(e0, wi_ref[...], preferred_element_type=jnp.float32)
                 + bi_ref[...]), axis=0, keepdims=True).reshape(1, 1, H)
    t1_ref[...] = jnp.sum(
        jnp.tanh(jnp.dot(e1, wi_ref[...], preferred_element_type=jnp.float32)
                 + bi_ref[...]), axis=0, keepdims=True).reshape(1, 1, H)


def _attention(feats0, g1r, g2r, W0, b0, att_i0, att_i1, W2, b2, Wint, bint):
    d0 = feats0.shape[1]
    full = lambda r, c: pl.BlockSpec((r, c), lambda i: (0, 0))
    out = _pc(
        _attn_body,
        grid=(NB,),
        in_specs=[
            pl.BlockSpec((BN, d0), lambda i: (i, 0)),
            pl.BlockSpec((BN, 20, 2 * H), lambda i: (i, 0, 0)),
            pl.BlockSpec((BN, 5, 256), lambda i: (i, 0, 0)),
            full(d0, H), full(1, H), full(1, H), full(1, 2 * H),
            full(1, H), full(1, H), full(256, H), full(1, H),
            full(H, H), full(1, H),
        ],
        out_specs=[
            pl.BlockSpec((BN, H), lambda i: (i, 0)),
            pl.BlockSpec((BN, H), lambda i: (i, 0)),
            pl.BlockSpec((1, 1, H), lambda i: (i, 0, 0)),
            pl.BlockSpec((1, 1, H), lambda i: (i, 0, 0)),
        ],
        out_shape=[
            jax.ShapeDtypeStruct((N0, H), jnp.float32),
            jax.ShapeDtypeStruct((N0, H), jnp.float32),
            jax.ShapeDtypeStruct((NB, 1, H), jnp.float32),
            jax.ShapeDtypeStruct((NB, 1, H), jnp.float32),
        ],
    )
    an0p = jnp.concatenate([att_i0[:, H:], jnp.zeros((1, H), jnp.float32)],
                           axis=1)
    return out(feats0, g1r, g2r, W0.T, b0.reshape(1, H),
               att_i0[:, :H], an0p, att_i1[:, :H], att_i1[:, H:],
               W2.T, b2.reshape(1, H), Wint.T, bint.reshape(1, H))


def _head_body(e0_ref, e1_ref, t0_ref, t1_ref, ai_ref, wp1_ref, bp1_ref,
               wp2_ref, bp2_ref, z_ref):
    s0 = jnp.sum(t0_ref[...], axis=0) * (1.0 / N0)
    s1 = jnp.sum(t1_ref[...], axis=0) * (1.0 / N0)
    l0 = jnp.sum(ai_ref[...] * s0)
    l1 = jnp.sum(ai_ref[...] * s1)
    m = jnp.maximum(l0, l1)
    x0 = jnp.exp(l0 - m)
    x1 = jnp.exp(l1 - m)
    inv = 1.0 / (x0 + x1)
    z = (x0 * inv) * e0_ref[...] + (x1 * inv) * e1_ref[...]
    u = _elu(jnp.dot(z, wp1_ref[...], preferred_element_type=jnp.float32)
             + bp1_ref[...])
    zp = jnp.dot(u, wp2_ref[...], preferred_element_type=jnp.float32) \
        + bp2_ref[...]
    nrm = jnp.sqrt(jnp.sum(zp * zp, axis=1, keepdims=True))
    z_ref[...] = zp / nrm


def _head(e0, e1, t0, t1, att_inter, Wp1, bp1, Wp2, bp2):
    full = lambda r, c: pl.BlockSpec((r, c), lambda i: (0, 0))
    return _pc(
        _head_body,
        grid=(NB,),
        in_specs=[
            pl.BlockSpec((BN, H), lambda i: (i, 0)),
            pl.BlockSpec((BN, H), lambda i: (i, 0)),
            pl.BlockSpec((NB, 1, H), lambda i: (0, 0, 0)),
            pl.BlockSpec((NB, 1, H), lambda i: (0, 0, 0)),
            full(1, H),
            full(H, H), full(1, H), full(H, H), full(1, H),
        ],
        out_specs=pl.BlockSpec((BN, H), lambda i: (i, 0)),
        out_shape=jax.ShapeDtypeStruct((N0, H), jnp.float32),
    )(e0, e1, t0, t1, att_inter, Wp1.T, bp1.reshape(1, H),
      Wp2.T, bp2.reshape(1, H))


def _sim_body(zi_ref, zall_ref, pos_ref, o_ref):
    i = pl.program_id(0)
    c = lax.dot_general(zi_ref[...], zall_ref[...],
                        (((1,), (1,)), ((), ())),
                        preferred_element_type=jnp.float32)
    e = jnp.exp(c * INV_TAU)
    num = jnp.sum(e * pos_ref[...], axis=1, keepdims=True)
    den = jnp.sum(e, axis=1, keepdims=True)
    part = -jnp.sum(jnp.log(num / (den + 1e-8))) * (1.0 / N0)

    @pl.when(i == 0)
    def _zero():
        o_ref[...] = jnp.zeros_like(o_ref)

    o_ref[...] += part


SB = 200


def _contrast(zhat, pos):
    return _pc(
        _sim_body,
        grid=(N0 // SB,),
        in_specs=[
            pl.BlockSpec((SB, H), lambda i: (i, 0)),
            pl.BlockSpec((N0, H), lambda i: (0, 0)),
            pl.BlockSpec((SB, N0), lambda i: (i, 0)),
        ],
        out_specs=pl.BlockSpec((1, 1), lambda i: (0, 0)),
        out_shape=jax.ShapeDtypeStruct((1, 1), jnp.float32),
    )(zhat, zhat, pos)


def kernel(feats0, feats1, feats2, pos, nei0, nei1, W0, b0, W1, b1, W2, b2,
           att_i0, att_i1, Wint, bint, att_inter, Wp1, bp1, Wp2, bp2):
    h1 = _project_h1(feats1, W1, b1)

    n0f = nei0.reshape(-1)
    n0p = jnp.concatenate([n0f, jnp.zeros((102400 - n0f.shape[0],),
                                          jnp.int32)])
    g1 = _sc_gather(h1, n0p, n_chunks=8, chunk=400)
    g1r = g1.reshape(5120, 20, 2 * H)

    n1f = nei1.reshape(-1)
    n1p = jnp.concatenate([n1f, jnp.zeros((25600 - n1f.shape[0],),
                                          jnp.int32)])
    g2 = _sc_gather(feats2, n1p, n_chunks=4, chunk=200)
    g2r = g2.reshape(5120, 5, 256)

    e0, e1, t0, t1 = _attention(feats0, g1r, g2r, W0, b0, att_i0, att_i1,
                                W2, b2, Wint, bint)
    zhat = _head(e0, e1, t0, t1, att_inter, Wp1, bp1, Wp2, bp2)
    out = _contrast(zhat, pos)
    return out[0, 0]


# speedup vs baseline: 1.6652x; 1.0251x over previous
import functools

import jax
import jax.numpy as jnp
from jax import lax
from jax.experimental import pallas as pl
from jax.experimental.pallas import tpu as pltpu
from jax.experimental.pallas import tpu_sc as plsc

_pc = pl.pallas_call

N0 = 5000
H = 64
INV_TAU = 2.0

NB = 5
BN = N0 // NB


def _elu(x):
    return jnp.where(x > 0, x, jnp.exp(x) - 1.0)


def _leaky(x):
    return jnp.where(x >= 0, x, 0.01 * x)


def _proj_body(x_ref, w_ref, b_ref, o_ref):
    h = _elu(
        jnp.dot(x_ref[...], w_ref[...], preferred_element_type=jnp.float32)
        + b_ref[...])
    o_ref[...] = jnp.concatenate([h, jnp.zeros_like(h)], axis=1)


def _project_h1(feats1, W1, b1):
    n, d = feats1.shape
    blk = 2000
    return _pc(
        _proj_body,
        grid=(n // blk,),
        in_specs=[
            pl.BlockSpec((blk, d), lambda i: (i, 0)),
            pl.BlockSpec((d, H), lambda i: (0, 0)),
            pl.BlockSpec((1, H), lambda i: (0, 0)),
        ],
        out_specs=pl.BlockSpec((blk, 2 * H), lambda i: (i, 0)),
        out_shape=jax.ShapeDtypeStruct((n, 2 * H), jnp.float32),
    )(feats1, W1.T, b1.reshape(1, H))


def _sc_gather(table, idx, n_chunks, chunk, nbuf=2):
    B = idx.shape[0]
    D = table.shape[1]
    info = plsc.get_sparse_core_info()
    NC, NS = info.num_cores, info.num_subcores
    per_w = B // (NC * NS)
    mesh = plsc.VectorSubcoreMesh(core_axis_name="c", subcore_axis_name="s")

    scratch = [pltpu.VMEM((per_w,), jnp.int32)]
    scratch += [pltpu.VMEM((chunk, D), jnp.float32) for _ in range(nbuf)]
    scratch += [pltpu.SemaphoreType.DMA for _ in range(2 * nbuf)]

    @functools.partial(
        pl.kernel,
        mesh=mesh,
        out_type=jax.ShapeDtypeStruct((B, D), jnp.float32),
        scratch_types=scratch,
    )
    def gk(idx_hbm, table_hbm, out_hbm, idx_v, *rest):
        bufs = rest[:nbuf]
        gsem = rest[nbuf:2 * nbuf]
        ssem = rest[2 * nbuf:3 * nbuf]
        wid = lax.axis_index("s") * NC + lax.axis_index("c")
        base = wid * per_w
        pltpu.sync_copy(idx_hbm.at[pl.ds(base, per_w)], idx_v)

        gh = [None] * n_chunks
        sh = [None] * n_chunks

        def issue(c):
            b = c % nbuf
            if c >= nbuf:
                sh[c - nbuf].wait()
            gh[c] = pltpu.async_copy(
                table_hbm.at[idx_v.at[pl.ds(c * chunk, chunk)]],
                bufs[b], gsem[b])

        for c in range(min(nbuf, n_chunks)):
            issue(c)
        for c in range(n_chunks):
            b = c % nbuf
            gh[c].wait()
            sh[c] = pltpu.async_copy(
                bufs[b], out_hbm.at[pl.ds(base + c * chunk, chunk)], ssem[b])
            if c + nbuf < n_chunks:
                issue(c + nbuf)
        for c in range(max(0, n_chunks - nbuf), n_chunks):
            sh[c].wait()

    return gk(idx, table)


def _attn_body(f0_ref, g1_ref, g2_ref, w0_ref, b0_ref, ar0_ref, an0_ref,
               ar1_ref, an1_ref, w2_ref, b2_ref, wi_ref, bi_ref,
               e0_ref, e1_ref, t0_ref, t1_ref):
    h0 = _elu(jnp.dot(f0_ref[...], w0_ref[...],
                      preferred_element_type=jnp.float32) + b0_ref[...])

    r0 = jnp.sum(h0 * ar0_ref[...], axis=1, keepdims=True)
    cols = [jnp.sum(g1_ref[:, s, :] * an0_ref[...], axis=1, keepdims=True)
            for s in range(20)]
    logit = _leaky(jnp.concatenate(cols, axis=1) + r0)
    m = jnp.max(logit, axis=1, keepdims=True)
    w = jnp.exp(logit - m)
    w = w / jnp.sum(w, axis=1, keepdims=True)
    acc = w[:, 0:1] * g1_ref[:, 0, :]
    for s in range(1, 20):
        acc = acc + w[:, s:s + 1] * g1_ref[:, s, :]
    e0 = _elu(acc[:, 0:H])
    e0_ref[...] = e0

    r1 = jnp.sum(h0 * ar1_ref[...], axis=1, keepdims=True)
    hs = [_elu(jnp.dot(g2_ref[:, s, :], w2_ref[...],
                       preferred_element_type=jnp.float32) + b2_ref[...])
          for s in range(5)]
    cols = [jnp.sum(h * an1_ref[...], axis=1, keepdims=True) for h in hs]
    logit = _leaky(jnp.concatenate(cols, axis=1) + r1)
    m = jnp.max(logit, axis=1, keepdims=True)
    w = jnp.exp(logit - m)
    w = w / jnp.sum(w, axis=1, keepdims=True)
    acc = w[:, 0:1] * hs[0]
    for s in range(1, 5):
        acc = acc + w[:, s:s + 1] * hs[s]
    e1 = _elu(acc)
    e1_ref[...] = e1

    t0_ref[...] = jnp.sum(
        jnp.tanh(jnp.dot(e0, wi_ref[...], preferred_element_type=jnp.float32)
                 + bi_ref[...]), axis=0, keepdims=True).reshape(1, 1, H)
    t1_ref[...] = jnp.sum(
        jnp.tanh(jnp.dot(e1, wi_ref[...], preferred_element_type=jnp.float32)
                 + bi_ref[...]), axis=0, keepdims=True).reshape(1, 1, H)


def _attention(feats0, g1r, g2r, W0, b0, att_i0, att_i1, W2, b2, Wint, bint):
    d0 = feats0.shape[1]
    full = lambda r, c: pl.BlockSpec((r, c), lambda i: (0, 0))
    out = _pc(
        _attn_body,
        grid=(NB,),
        in_specs=[
            pl.BlockSpec((BN, d0), lambda i: (i, 0)),
            pl.BlockSpec((BN, 20, 2 * H), lambda i: (i, 0, 0)),
            pl.BlockSpec((BN, 5, 256), lambda i: (i, 0, 0)),
            full(d0, H), full(1, H), full(1, H), full(1, 2 * H),
            full(1, H), full(1, H), full(256, H), full(1, H),
            full(H, H), full(1, H),
        ],
        out_specs=[
            pl.BlockSpec((BN, H), lambda i: (i, 0)),
            pl.BlockSpec((BN, H), lambda i: (i, 0)),
            pl.BlockSpec((1, 1, H), lambda i: (i, 0, 0)),
            pl.BlockSpec((1, 1, H), lambda i: (i, 0, 0)),
        ],
        out_shape=[
            jax.ShapeDtypeStruct((N0, H), jnp.float32),
            jax.ShapeDtypeStruct((N0, H), jnp.float32),
            jax.ShapeDtypeStruct((NB, 1, H), jnp.float32),
            jax.ShapeDtypeStruct((NB, 1, H), jnp.float32),
        ],
    )
    an0p = jnp.concatenate([att_i0[:, H:], jnp.zeros((1, H), jnp.float32)],
                           axis=1)
    return out(feats0, g1r, g2r, W0.T, b0.reshape(1, H),
               att_i0[:, :H], an0p, att_i1[:, :H], att_i1[:, H:],
               W2.T, b2.reshape(1, H), Wint.T, bint.reshape(1, H))


def _head_body(e0_ref, e1_ref, t0_ref, t1_ref, ai_ref, wp1_ref, bp1_ref,
               wp2_ref, bp2_ref, z_ref):
    s0 = jnp.sum(t0_ref[...], axis=0) * (1.0 / N0)
    s1 = jnp.sum(t1_ref[...], axis=0) * (1.0 / N0)
    l0 = jnp.sum(ai_ref[...] * s0)
    l1 = jnp.sum(ai_ref[...] * s1)
    m = jnp.maximum(l0, l1)
    x0 = jnp.exp(l0 - m)
    x1 = jnp.exp(l1 - m)
    inv = 1.0 / (x0 + x1)
    z = (x0 * inv) * e0_ref[...] + (x1 * inv) * e1_ref[...]
    u = _elu(jnp.dot(z, wp1_ref[...], preferred_element_type=jnp.float32)
             + bp1_ref[...])
    zp = jnp.dot(u, wp2_ref[...], preferred_element_type=jnp.float32) \
        + bp2_ref[...]
    nrm = jnp.sqrt(jnp.sum(zp * zp, axis=1, keepdims=True))
    z_ref[...] = zp / nrm


def _head(e0, e1, t0, t1, att_inter, Wp1, bp1, Wp2, bp2):
    full = lambda r, c: pl.BlockSpec((r, c), lambda i: (0, 0))
    return _pc(
        _head_body,
        grid=(NB,),
        in_specs=[
            pl.BlockSpec((BN, H), lambda i: (i, 0)),
            pl.BlockSpec((BN, H), lambda i: (i, 0)),
            pl.BlockSpec((NB, 1, H), lambda i: (0, 0, 0)),
            pl.BlockSpec((NB, 1, H), lambda i: (0, 0, 0)),
            full(1, H),
            full(H, H), full(1, H), full(H, H), full(1, H),
        ],
        out_specs=pl.BlockSpec((BN, H), lambda i: (i, 0)),
        out_shape=jax.ShapeDtypeStruct((N0, H), jnp.float32),
    )(e0, e1, t0, t1, att_inter, Wp1.T, bp1.reshape(1, H),
      Wp2.T, bp2.reshape(1, H))


def _sim_body(zi_ref, zall_ref, pos_ref, o_ref):
    i = pl.program_id(0)
    c = lax.dot_general(zi_ref[...], zall_ref[...],
                        (((1,), (1,)), ((), ())),
                        preferred_element_type=jnp.float32)
    e = jnp.exp(c * INV_TAU)
    num = jnp.sum(e * pos_ref[...], axis=1, keepdims=True)
    den = jnp.sum(e, axis=1, keepdims=True)
    part = -jnp.sum(jnp.log(num / (den + 1e-8))) * (1.0 / N0)

    @pl.when(i == 0)
    def _zero():
        o_ref[...] = jnp.zeros_like(o_ref)

    o_ref[...] += part


SB = 200


def _contrast(zhat, pos):
    return _pc(
        _sim_body,
        grid=(N0 // SB,),
        in_specs=[
            pl.BlockSpec((SB, H), lambda i: (i, 0)),
            pl.BlockSpec((N0, H), lambda i: (0, 0)),
            pl.BlockSpec((SB, N0), lambda i: (i, 0)),
        ],
        out_specs=pl.BlockSpec((1, 1), lambda i: (0, 0)),
        out_shape=jax.ShapeDtypeStruct((1, 1), jnp.float32),
    )(zhat, zhat, pos)


def kernel(feats0, feats1, feats2, pos, nei0, nei1, W0, b0, W1, b1, W2, b2,
           att_i0, att_i1, Wint, bint, att_inter, Wp1, bp1, Wp2, bp2):
    h1 = _project_h1(feats1, W1, b1)

    n0f = nei0.reshape(-1)
    n0p = jnp.concatenate([n0f, jnp.zeros((102400 - n0f.shape[0],),
                                          jnp.int32)])
    g1 = _sc_gather(h1, n0p, n_chunks=8, chunk=400)
    g1r = g1.reshape(5120, 20, 2 * H)

    n1f = nei1.reshape(-1)
    n1p = jnp.concatenate([n1f, jnp.zeros((25600 - n1f.shape[0],),
                                          jnp.int32)])
    g2 = _sc_gather(feats2, n1p, n_chunks=4, chunk=200)
    g2r = g2.reshape(5120, 5, 256)

    e0, e1, t0, t1 = _attention(feats0, g1r, g2r, W0, b0, att_i0, att_i1,
                                W2, b2, Wint, bint)
    zhat = _head(e0, e1, t0, t1, att_inter, Wp1, bp1, Wp2, bp2)
    out = _contrast(zhat, pos)
    return out[0, 0]


# speedup vs baseline: 2.2544x; 1.3538x over previous
import functools

import jax
import jax.numpy as jnp
from jax import lax
from jax.experimental import pallas as pl
from jax.experimental.pallas import tpu as pltpu
from jax.experimental.pallas import tpu_sc as plsc

_pc = pl.pallas_call

N0 = 5000
H = 64
INV_TAU = 2.0

NB = 5
BN = N0 // NB


def _elu(x):
    return jnp.where(x > 0, x, jnp.exp(x) - 1.0)


def _leaky(x):
    return jnp.where(x >= 0, x, 0.01 * x)


def _proj_body(x_ref, w_ref, b_ref, o_ref):
    h = _elu(
        jnp.dot(x_ref[...], w_ref[...], preferred_element_type=jnp.float32)
        + b_ref[...])
    o_ref[...] = jnp.concatenate([h, jnp.zeros_like(h)], axis=1)


def _project_h1(feats1, W1, b1):
    n, d = feats1.shape
    blk = 2000
    return _pc(
        _proj_body,
        grid=(n // blk,),
        in_specs=[
            pl.BlockSpec((blk, d), lambda i: (i, 0)),
            pl.BlockSpec((d, H), lambda i: (0, 0)),
            pl.BlockSpec((1, H), lambda i: (0, 0)),
        ],
        out_specs=pl.BlockSpec((blk, 2 * H), lambda i: (i, 0)),
        out_shape=jax.ShapeDtypeStruct((n, 2 * H), jnp.float32),
    )(feats1, W1.T, b1.reshape(1, H))


def _sc_gathers(idx1, table1, idx2, table2):
    cfg = []
    info = plsc.get_sparse_core_info()
    NC, NS = info.num_cores, info.num_subcores
    NW = NC * NS
    for idx, table, chunk, nbuf in ((idx1, table1, 160, 3),
                                    (idx2, table2, 80, 2)):
        B = idx.shape[0]
        D = table.shape[1]
        per_w = B // NW
        cfg.append((B, D, per_w, chunk, per_w // chunk, nbuf))
    mesh = plsc.VectorSubcoreMesh(core_axis_name="c", subcore_axis_name="s",
                                  num_cores=NC)

    scratch = []
    for (B, D, per_w, chunk, n_chunks, nbuf) in cfg:
        scratch.append(pltpu.VMEM((per_w,), jnp.int32))
        scratch += [pltpu.VMEM((chunk, D), jnp.float32) for _ in range(nbuf)]
        scratch += [pltpu.SemaphoreType.DMA for _ in range(2 * nbuf)]

    @functools.partial(
        pl.kernel,
        mesh=mesh,
        out_type=[jax.ShapeDtypeStruct((cfg[0][0], cfg[0][1]), jnp.float32),
                  jax.ShapeDtypeStruct((cfg[1][0], cfg[1][1]), jnp.float32)],
        scratch_types=scratch,
    )
    def gk(idx1_hbm, t1_hbm, idx2_hbm, t2_hbm, out1_hbm, out2_hbm, *rest):
        wid = lax.axis_index("s") * NC + lax.axis_index("c")
        p = 0
        for k, (idx_hbm, table_hbm, out_hbm) in enumerate(
                ((idx1_hbm, t1_hbm, out1_hbm), (idx2_hbm, t2_hbm, out2_hbm))):
            (B, D, per_w, chunk, n_chunks, nbuf) = cfg[k]
            idx_v = rest[p]
            bufs = rest[p + 1:p + 1 + nbuf]
            gsem = rest[p + 1 + nbuf:p + 1 + 2 * nbuf]
            ssem = rest[p + 1 + 2 * nbuf:p + 1 + 3 * nbuf]
            p += 1 + 3 * nbuf
            base = wid * per_w
            pltpu.sync_copy(idx_hbm.at[pl.ds(base, per_w)], idx_v)

            gh = [None] * n_chunks
            sh = [None] * n_chunks

            def issue(c, bufs=bufs, gsem=gsem, sh=sh, gh=gh,
                      table_hbm=table_hbm, idx_v=idx_v, chunk=chunk,
                      nbuf=nbuf):
                b = c % nbuf
                if c >= nbuf:
                    sh[c - nbuf].wait()
                gh[c] = pltpu.async_copy(
                    table_hbm.at[idx_v.at[pl.ds(c * chunk, chunk)]],
                    bufs[b], gsem[b])

            for c in range(min(nbuf, n_chunks)):
                issue(c)
            for c in range(n_chunks):
                b = c % nbuf
                gh[c].wait()
                sh[c] = pltpu.async_copy(
                    bufs[b], out_hbm.at[pl.ds(base + c * chunk, chunk)],
                    ssem[b])
                if c + nbuf < n_chunks:
                    issue(c + nbuf)
            for c in range(max(0, n_chunks - nbuf), n_chunks):
                sh[c].wait()

    return gk(idx1, table1, idx2, table2)


SB = 200
NC_STEPS = N0 // SB
NBA = 3
NBB = 2


def _attn_block(f0, g1s, g2s, w0, b0, ar0, an0, ar1, an1, w2, b2, wi, bi):
    h0 = _elu(jnp.dot(f0, w0, preferred_element_type=jnp.float32) + b0)

    r0 = jnp.sum(h0 * ar0, axis=1, keepdims=True)
    cols = [jnp.sum(g * an0, axis=1, keepdims=True) for g in g1s]
    logit = _leaky(jnp.concatenate(cols, axis=1) + r0)
    m = jnp.max(logit, axis=1, keepdims=True)
    w = jnp.exp(logit - m)
    w = w / jnp.sum(w, axis=1, keepdims=True)
    acc = w[:, 0:1] * g1s[0]
    for t in range(1, len(g1s)):
        acc = acc + w[:, t:t + 1] * g1s[t]
    e0 = _elu(acc[:, 0:H])

    r1 = jnp.sum(h0 * ar1, axis=1, keepdims=True)
    hs = [_elu(jnp.dot(g, w2, preferred_element_type=jnp.float32) + b2)
          for g in g2s]
    cols = [jnp.sum(h * an1, axis=1, keepdims=True) for h in hs]
    logit = _leaky(jnp.concatenate(cols, axis=1) + r1)
    m = jnp.max(logit, axis=1, keepdims=True)
    w = jnp.exp(logit - m)
    w = w / jnp.sum(w, axis=1, keepdims=True)
    acc = w[:, 0:1] * hs[0]
    for t in range(1, len(hs)):
        acc = acc + w[:, t:t + 1] * hs[t]
    e1 = _elu(acc)

    p0 = jnp.sum(jnp.tanh(jnp.dot(e0, wi, preferred_element_type=jnp.float32)
                          + bi), axis=0, keepdims=True)
    p1 = jnp.sum(jnp.tanh(jnp.dot(e1, wi, preferred_element_type=jnp.float32)
                          + bi), axis=0, keepdims=True)
    return e0, e1, p0, p1


def _attn_a_body(f0_ref, g1_ref, g2_ref,
                 w0_ref, b0_ref, ar0_ref, an0_ref, ar1_ref, an1_ref,
                 w2_ref, b2_ref, wi_ref, bi_ref,
                 e0_ref, e1_ref, t0_ref, t1_ref):
    e0, e1, p0, p1 = _attn_block(
        f0_ref[...], [g1_ref[t] for t in range(20)],
        [g2_ref[t] for t in range(5)],
        w0_ref[...], b0_ref[...], ar0_ref[...], an0_ref[...],
        ar1_ref[...], an1_ref[...], w2_ref[...], b2_ref[...],
        wi_ref[...], bi_ref[...])
    e0_ref[...] = e0
    e1_ref[...] = e1
    t0_ref[...] = p0.reshape(1, 1, H)
    t1_ref[...] = p1.reshape(1, 1, H)


def _attn_a(feats0, g1r, g2r, W0T, b0r, ar0, an0p, ar1, an1, W2T, b2r,
            WintT, bintr):
    d0 = feats0.shape[1]
    full = lambda r, c: pl.BlockSpec((r, c), lambda i: (0, 0))
    return _pc(
        _attn_a_body,
        grid=(NBA,),
        in_specs=[
            pl.BlockSpec((BN, d0), lambda i: (i, 0)),
            pl.BlockSpec((20, BN, 2 * H), lambda i: (0, i, 0)),
            pl.BlockSpec((5, BN, 256), lambda i: (0, i, 0)),
            full(d0, H), full(1, H), full(1, H), full(1, 2 * H),
            full(1, H), full(1, H), full(256, H), full(1, H),
            full(H, H), full(1, H),
        ],
        out_specs=[
            pl.BlockSpec((BN, H), lambda i: (i, 0)),
            pl.BlockSpec((BN, H), lambda i: (i, 0)),
            pl.BlockSpec((1, 1, H), lambda i: (i, 0, 0)),
            pl.BlockSpec((1, 1, H), lambda i: (i, 0, 0)),
        ],
        out_shape=[
            jax.ShapeDtypeStruct((NBA * BN, H), jnp.float32),
            jax.ShapeDtypeStruct((NBA * BN, H), jnp.float32),
            jax.ShapeDtypeStruct((NBA, 1, H), jnp.float32),
            jax.ShapeDtypeStruct((NBA, 1, H), jnp.float32),
        ],
    )(feats0, g1r, g2r, W0T, b0r, ar0, an0p, ar1, an1, W2T, b2r,
      WintT, bintr)


def _fused_b_body(f0_ref, g1_ref, g2_ref, e0a_ref, e1a_ref, ta0_ref, ta1_ref,
                  pos_ref,
                  w0_ref, b0_ref, ar0_ref, an0_ref, ar1_ref, an1_ref,
                  w2_ref, b2_ref, wi_ref, bi_ref, ai_ref,
                  wp1_ref, bp1_ref, wp2_ref, bp2_ref,
                  o_ref, e0s, e1s, zs, t0s, t1s):
    i = pl.program_id(0)

    @pl.when(i < NBB)
    def _attn_b():
        e0, e1, p0, p1 = _attn_block(
            f0_ref[...], [g1_ref[t] for t in range(20)],
            [g2_ref[t] for t in range(5)],
            w0_ref[...], b0_ref[...], ar0_ref[...], an0_ref[...],
            ar1_ref[...], an1_ref[...], w2_ref[...], b2_ref[...],
            wi_ref[...], bi_ref[...])
        e0s[pl.ds(i * BN, BN), :] = e0
        e1s[pl.ds(i * BN, BN), :] = e1
        zero = jnp.zeros_like(p0)
        t0s[...] = jnp.where(i == 0, zero, t0s[...]) + p0
        t1s[...] = jnp.where(i == 0, zero, t1s[...]) + p1

    @pl.when(jnp.logical_and(i >= NBB, i < NBB + NB))
    def _head():
        b = i - NBB
        t0 = jnp.sum(ta0_ref[...], axis=0) + t0s[...]
        t1 = jnp.sum(ta1_ref[...], axis=0) + t1s[...]
        l0 = jnp.sum(ai_ref[...] * t0) * (1.0 / N0)
        l1 = jnp.sum(ai_ref[...] * t1) * (1.0 / N0)
        m = jnp.maximum(l0, l1)
        x0 = jnp.exp(l0 - m)
        x1 = jnp.exp(l1 - m)
        inv = 1.0 / (x0 + x1)

        def _project(z):
            u = _elu(jnp.dot(z, wp1_ref[...],
                             preferred_element_type=jnp.float32)
                     + bp1_ref[...])
            zp = jnp.dot(u, wp2_ref[...],
                         preferred_element_type=jnp.float32) + bp2_ref[...]
            nrm = jnp.sqrt(jnp.sum(zp * zp, axis=1, keepdims=True))
            zs[pl.ds(b * BN, BN), :] = zp / nrm

        @pl.when(b < NBA)
        def _from_a():
            _project((x0 * inv) * e0a_ref[...] + (x1 * inv) * e1a_ref[...])

        @pl.when(b >= NBA)
        def _from_b():
            lb = jnp.maximum(b - NBA, 0)
            _project((x0 * inv) * e0s[pl.ds(lb * BN, BN), :]
                     + (x1 * inv) * e1s[pl.ds(lb * BN, BN), :])

    @pl.when(i >= NBB + NB)
    def _contrast():
        k = i - (NBB + NB)
        zi = zs[pl.ds(k * SB, SB), :]
        c = lax.dot_general(zi, zs[...], (((1,), (1,)), ((), ())),
                            preferred_element_type=jnp.float32)
        e = jnp.exp(c * INV_TAU)
        num = jnp.sum(e * pos_ref[...], axis=1, keepdims=True)
        den = jnp.sum(e, axis=1, keepdims=True)
        part = -jnp.sum(jnp.log(num / (den + 1e-8))) * (1.0 / N0)
        zero = jnp.zeros_like(o_ref[...])
        o_ref[...] = jnp.where(i == NBB + NB, zero, o_ref[...]) + part


def _fused_b(feats0, g1r, g2r, e0a, e1a, ta0, ta1, pos,
             W0T, b0r, ar0, an0p, ar1, an1, W2T, b2r, WintT, bintr,
             att_inter, Wp1T, bp1r, Wp2T, bp2r):
    d0 = feats0.shape[1]
    full = lambda r, c: pl.BlockSpec((r, c), lambda i: (0, 0))
    battn = lambda i: jnp.clip(i, 0, NBB - 1)
    call = _pc(
        _fused_b_body,
        grid=(NBB + NB + NC_STEPS,),
        in_specs=[
            pl.BlockSpec((BN, d0), lambda i: (battn(i) + NBA, 0)),
            pl.BlockSpec((20, BN, 2 * H), lambda i: (0, battn(i), 0)),
            pl.BlockSpec((5, BN, 256), lambda i: (0, battn(i), 0)),
            pl.BlockSpec((BN, H), lambda i: (jnp.clip(i - NBB, 0, NBA - 1),
                                             0)),
            pl.BlockSpec((BN, H), lambda i: (jnp.clip(i - NBB, 0, NBA - 1),
                                             0)),
            pl.BlockSpec((NBA, 1, H), lambda i: (0, 0, 0)),
            pl.BlockSpec((NBA, 1, H), lambda i: (0, 0, 0)),
            pl.BlockSpec((SB, N0),
                         lambda i: (jnp.clip(i - NBB - NB, 0,
                                             NC_STEPS - 1), 0)),
            full(d0, H), full(1, H), full(1, H), full(1, 2 * H),
            full(1, H), full(1, H), full(256, H), full(1, H),
            full(H, H), full(1, H), full(1, H),
            full(H, H), full(1, H), full(H, H), full(1, H),
        ],
        out_specs=pl.BlockSpec((1, 1), lambda i: (0, 0)),
        out_shape=jax.ShapeDtypeStruct((1, 1), jnp.float32),
        scratch_shapes=[
            pltpu.VMEM((NBB * BN, H), jnp.float32),
            pltpu.VMEM((NBB * BN, H), jnp.float32),
            pltpu.VMEM((N0, H), jnp.float32),
            pltpu.VMEM((1, H), jnp.float32),
            pltpu.VMEM((1, H), jnp.float32),
        ],
    )
    return call(feats0, g1r, g2r, e0a, e1a, ta0, ta1, pos,
                W0T, b0r, ar0, an0p, ar1, an1, W2T, b2r, WintT, bintr,
                att_inter, Wp1T, bp1r, Wp2T, bp2r)


def kernel(feats0, feats1, feats2, pos, nei0, nei1, W0, b0, W1, b1, W2, b2,
           att_i0, att_i1, Wint, bint, att_inter, Wp1, bp1, Wp2, bp2):
    h1 = _project_h1(feats1, W1, b1)

    nT0 = nei0.T
    nT1 = nei1.T
    z20 = jnp.zeros((20, 72), jnp.int32)
    z5 = jnp.zeros((5, 72), jnp.int32)
    n0A = jnp.concatenate([nT0[:, :3000], z20], axis=1).reshape(-1)
    n1A = jnp.concatenate([nT1[:, :3000], z5], axis=1).reshape(-1)
    n0B = jnp.concatenate([nT0[:, 3000:], z20[:, :48]], axis=1).reshape(-1)
    n1B = jnp.concatenate([nT1[:, 3000:], z5[:, :48]], axis=1).reshape(-1)

    g1A, g2A = _sc_gathers(n0A, h1, n1A, feats2)
    g1B, g2B = _sc_gathers(n0B, h1, n1B, feats2)

    W0T = W0.T
    b0r = b0.reshape(1, H)
    ar0 = att_i0[:, :H]
    an0p = jnp.concatenate([att_i0[:, H:], jnp.zeros((1, H), jnp.float32)],
                           axis=1)
    ar1 = att_i1[:, :H]
    an1 = att_i1[:, H:]
    W2T = W2.T
    b2r = b2.reshape(1, H)
    WintT = Wint.T
    bintr = bint.reshape(1, H)

    e0A, e1A, tA0, tA1 = _attn_a(
        feats0, g1A.reshape(20, 3072, 2 * H), g2A.reshape(5, 3072, 256),
        W0T, b0r, ar0, an0p, ar1, an1, W2T, b2r, WintT, bintr)

    out = _fused_b(
        feats0, g1B.reshape(20, 2048, 2 * H), g2B.reshape(5, 2048, 256),
        e0A, e1A, tA0, tA1, pos,
        W0T, b0r, ar0, an0p, ar1, an1, W2T, b2r, WintT, bintr,
        att_inter, Wp1.T, bp1.reshape(1, H), Wp2.T, bp2.reshape(1, H))
    return out[0, 0]


# speedup vs baseline: 2.2795x; 1.0111x over previous
import functools

import jax
import jax.numpy as jnp
from jax import lax
from jax.experimental import pallas as pl
from jax.experimental.pallas import tpu as pltpu
from jax.experimental.pallas import tpu_sc as plsc

_pc = pl.pallas_call

N0 = 5000
H = 64
INV_TAU = 2.0

NB = 5
BN = N0 // NB


def _elu(x):
    return jnp.where(x > 0, x, jnp.exp(x) - 1.0)


def _leaky(x):
    return jnp.where(x >= 0, x, 0.01 * x)


def _proj_body(x_ref, w_ref, b_ref, o_ref):
    h = _elu(
        jnp.dot(x_ref[...], w_ref[...], preferred_element_type=jnp.float32)
        + b_ref[...])
    o_ref[...] = jnp.concatenate([h, jnp.zeros_like(h)], axis=1)


def _project_h1(feats1, W1, b1):
    n, d = feats1.shape
    blk = 2000
    return _pc(
        _proj_body,
        grid=(n // blk,),
        in_specs=[
            pl.BlockSpec((blk, d), lambda i: (i, 0)),
            pl.BlockSpec((d, H), lambda i: (0, 0)),
            pl.BlockSpec((1, H), lambda i: (0, 0)),
        ],
        out_specs=pl.BlockSpec((blk, 2 * H), lambda i: (i, 0)),
        out_shape=jax.ShapeDtypeStruct((n, 2 * H), jnp.float32),
    )(feats1, W1.T, b1.reshape(1, H))


def _sc_gathers(idx1, table1, idx2, table2):
    cfg = []
    info = plsc.get_sparse_core_info()
    NC, NS = info.num_cores, info.num_subcores
    NW = NC * NS
    for idx, table, chunk, nbuf in ((idx1, table1, 160, 3),
                                    (idx2, table2, 80, 2)):
        B = idx.shape[0]
        D = table.shape[1]
        per_w = B // NW
        cfg.append((B, D, per_w, chunk, per_w // chunk, nbuf))
    mesh = plsc.VectorSubcoreMesh(core_axis_name="c", subcore_axis_name="s",
                                  num_cores=NC)

    scratch = []
    for (B, D, per_w, chunk, n_chunks, nbuf) in cfg:
        scratch.append(pltpu.VMEM((per_w,), jnp.int32))
        scratch += [pltpu.VMEM((chunk, D), jnp.float32) for _ in range(nbuf)]
        scratch += [pltpu.SemaphoreType.DMA for _ in range(2 * nbuf)]

    @functools.partial(
        pl.kernel,
        mesh=mesh,
        out_type=[jax.ShapeDtypeStruct((cfg[0][0], cfg[0][1]), jnp.float32),
                  jax.ShapeDtypeStruct((cfg[1][0], cfg[1][1]), jnp.float32)],
        scratch_types=scratch,
    )
    def gk(idx1_hbm, t1_hbm, idx2_hbm, t2_hbm, out1_hbm, out2_hbm, *rest):
        wid = lax.axis_index("s") * NC + lax.axis_index("c")
        p = 0
        for k, (idx_hbm, table_hbm, out_hbm) in enumerate(
                ((idx1_hbm, t1_hbm, out1_hbm), (idx2_hbm, t2_hbm, out2_hbm))):
            (B, D, per_w, chunk, n_chunks, nbuf) = cfg[k]
            idx_v = rest[p]
            bufs = rest[p + 1:p + 1 + nbuf]
            gsem = rest[p + 1 + nbuf:p + 1 + 2 * nbuf]
            ssem = rest[p + 1 + 2 * nbuf:p + 1 + 3 * nbuf]
            p += 1 + 3 * nbuf
            base = wid * per_w
            pltpu.sync_copy(idx_hbm.at[pl.ds(base, per_w)], idx_v)

            gh = [None] * n_chunks
            sh = [None] * n_chunks

            def issue(c, bufs=bufs, gsem=gsem, sh=sh, gh=gh,
                      table_hbm=table_hbm, idx_v=idx_v, chunk=chunk,
                      nbuf=nbuf):
                b = c % nbuf
                if c >= nbuf:
                    sh[c - nbuf].wait()
                gh[c] = pltpu.async_copy(
                    table_hbm.at[idx_v.at[pl.ds(c * chunk, chunk)]],
                    bufs[b], gsem[b])

            for c in range(min(nbuf, n_chunks)):
                issue(c)
            for c in range(n_chunks):
                b = c % nbuf
                gh[c].wait()
                sh[c] = pltpu.async_copy(
                    bufs[b], out_hbm.at[pl.ds(base + c * chunk, chunk)],
                    ssem[b])
                if c + nbuf < n_chunks:
                    issue(c + nbuf)
            for c in range(max(0, n_chunks - nbuf), n_chunks):
                sh[c].wait()

    return gk(idx1, table1, idx2, table2)


SB = 200
NC_STEPS = N0 // SB
NBA = 4
NBB = 1


def _attn_block(f0, g1s, g2s, w0, b0, ar0, an0, ar1, an1, w2, b2, wi, bi):
    h0 = _elu(jnp.dot(f0, w0, preferred_element_type=jnp.float32) + b0)

    r0 = jnp.sum(h0 * ar0, axis=1, keepdims=True)
    cols = [jnp.sum(g * an0, axis=1, keepdims=True) for g in g1s]
    logit = _leaky(jnp.concatenate(cols, axis=1) + r0)
    m = jnp.max(logit, axis=1, keepdims=True)
    w = jnp.exp(logit - m)
    w = w / jnp.sum(w, axis=1, keepdims=True)
    acc = w[:, 0:1] * g1s[0]
    for t in range(1, len(g1s)):
        acc = acc + w[:, t:t + 1] * g1s[t]
    e0 = _elu(acc[:, 0:H])

    r1 = jnp.sum(h0 * ar1, axis=1, keepdims=True)
    hs = [_elu(jnp.dot(g, w2, preferred_element_type=jnp.float32) + b2)
          for g in g2s]
    cols = [jnp.sum(h * an1, axis=1, keepdims=True) for h in hs]
    logit = _leaky(jnp.concatenate(cols, axis=1) + r1)
    m = jnp.max(logit, axis=1, keepdims=True)
    w = jnp.exp(logit - m)
    w = w / jnp.sum(w, axis=1, keepdims=True)
    acc = w[:, 0:1] * hs[0]
    for t in range(1, len(hs)):
        acc = acc + w[:, t:t + 1] * hs[t]
    e1 = _elu(acc)

    p0 = jnp.sum(jnp.tanh(jnp.dot(e0, wi, preferred_element_type=jnp.float32)
                          + bi), axis=0, keepdims=True)
    p1 = jnp.sum(jnp.tanh(jnp.dot(e1, wi, preferred_element_type=jnp.float32)
                          + bi), axis=0, keepdims=True)
    return e0, e1, p0, p1


def _attn_a_body(f0_ref, g1_ref, g2_ref,
                 w0_ref, b0_ref, ar0_ref, an0_ref, ar1_ref, an1_ref,
                 w2_ref, b2_ref, wi_ref, bi_ref,
                 e0_ref, e1_ref, t0_ref, t1_ref):
    e0, e1, p0, p1 = _attn_block(
        f0_ref[...], [g1_ref[t] for t in range(20)],
        [g2_ref[t] for t in range(5)],
        w0_ref[...], b0_ref[...], ar0_ref[...], an0_ref[...],
        ar1_ref[...], an1_ref[...], w2_ref[...], b2_ref[...],
        wi_ref[...], bi_ref[...])
    e0_ref[...] = e0
    e1_ref[...] = e1
    t0_ref[...] = p0.reshape(1, 1, H)
    t1_ref[...] = p1.reshape(1, 1, H)


def _attn_a(feats0, g1r, g2r, W0T, b0r, ar0, an0p, ar1, an1, W2T, b2r,
            WintT, bintr):
    d0 = feats0.shape[1]
    full = lambda r, c: pl.BlockSpec((r, c), lambda i: (0, 0))
    return _pc(
        _attn_a_body,
        grid=(NBA,),
        in_specs=[
            pl.BlockSpec((BN, d0), lambda i: (i, 0)),
            pl.BlockSpec((20, BN, 2 * H), lambda i: (0, i, 0)),
            pl.BlockSpec((5, BN, 256), lambda i: (0, i, 0)),
            full(d0, H), full(1, H), full(1, H), full(1, 2 * H),
            full(1, H), full(1, H), full(256, H), full(1, H),
            full(H, H), full(1, H),
        ],
        out_specs=[
            pl.BlockSpec((BN, H), lambda i: (i, 0)),
            pl.BlockSpec((BN, H), lambda i: (i, 0)),
            pl.BlockSpec((1, 1, H), lambda i: (i, 0, 0)),
            pl.BlockSpec((1, 1, H), lambda i: (i, 0, 0)),
        ],
        out_shape=[
            jax.ShapeDtypeStruct((NBA * BN, H), jnp.float32),
            jax.ShapeDtypeStruct((NBA * BN, H), jnp.float32),
            jax.ShapeDtypeStruct((NBA, 1, H), jnp.float32),
            jax.ShapeDtypeStruct((NBA, 1, H), jnp.float32),
        ],
    )(feats0, g1r, g2r, W0T, b0r, ar0, an0p, ar1, an1, W2T, b2r,
      WintT, bintr)


def _fused_b_body(f0_ref, g1_ref, g2_ref, e0a_ref, e1a_ref, ta0_ref, ta1_ref,
                  pos_ref,
                  w0_ref, b0_ref, ar0_ref, an0_ref, ar1_ref, an1_ref,
                  w2_ref, b2_ref, wi_ref, bi_ref, ai_ref,
                  wp1_ref, bp1_ref, wp2_ref, bp2_ref,
                  o_ref, e0s, e1s, zs, t0s, t1s):
    i = pl.program_id(0)

    @pl.when(i < NBB)
    def _attn_b():
        e0, e1, p0, p1 = _attn_block(
            f0_ref[...], [g1_ref[t] for t in range(20)],
            [g2_ref[t] for t in range(5)],
            w0_ref[...], b0_ref[...], ar0_ref[...], an0_ref[...],
            ar1_ref[...], an1_ref[...], w2_ref[...], b2_ref[...],
            wi_ref[...], bi_ref[...])
        e0s[pl.ds(i * BN, BN), :] = e0
        e1s[pl.ds(i * BN, BN), :] = e1
        zero = jnp.zeros_like(p0)
        t0s[...] = jnp.where(i == 0, zero, t0s[...]) + p0
        t1s[...] = jnp.where(i == 0, zero, t1s[...]) + p1

    @pl.when(jnp.logical_and(i >= NBB, i < NBB + NB))
    def _head():
        b = i - NBB
        t0 = jnp.sum(ta0_ref[...], axis=0) + t0s[...]
        t1 = jnp.sum(ta1_ref[...], axis=0) + t1s[...]
        l0 = jnp.sum(ai_ref[...] * t0) * (1.0 / N0)
        l1 = jnp.sum(ai_ref[...] * t1) * (1.0 / N0)
        m = jnp.maximum(l0, l1)
        x0 = jnp.exp(l0 - m)
        x1 = jnp.exp(l1 - m)
        inv = 1.0 / (x0 + x1)

        def _project(z):
            u = _elu(jnp.dot(z, wp1_ref[...],
                             preferred_element_type=jnp.float32)
                     + bp1_ref[...])
            zp = jnp.dot(u, wp2_ref[...],
                         preferred_element_type=jnp.float32) + bp2_ref[...]
            nrm = jnp.sqrt(jnp.sum(zp * zp, axis=1, keepdims=True))
            zs[pl.ds(b * BN, BN), :] = zp / nrm

        @pl.when(b < NBA)
        def _from_a():
            _project((x0 * inv) * e0a_ref[...] + (x1 * inv) * e1a_ref[...])

        @pl.when(b >= NBA)
        def _from_b():
            lb = jnp.maximum(b - NBA, 0)
            _project((x0 * inv) * e0s[pl.ds(lb * BN, BN), :]
                     + (x1 * inv) * e1s[pl.ds(lb * BN, BN), :])

    @pl.when(i >= NBB + NB)
    def _contrast():
        k = i - (NBB + NB)
        zi = zs[pl.ds(k * SB, SB), :]
        c = lax.dot_general(zi, zs[...], (((1,), (1,)), ((), ())),
                            preferred_element_type=jnp.float32)
        e = jnp.exp(c * INV_TAU)
        num = jnp.sum(e * pos_ref[...], axis=1, keepdims=True)
        den = jnp.sum(e, axis=1, keepdims=True)
        part = -jnp.sum(jnp.log(num / (den + 1e-8))) * (1.0 / N0)
        zero = jnp.zeros_like(o_ref[...])
        o_ref[...] = jnp.where(i == NBB + NB, zero, o_ref[...]) + part


def _fused_b(feats0, g1r, g2r, e0a, e1a, ta0, ta1, pos,
             W0T, b0r, ar0, an0p, ar1, an1, W2T, b2r, WintT, bintr,
             att_inter, Wp1T, bp1r, Wp2T, bp2r):
    d0 = feats0.shape[1]
    full = lambda r, c: pl.BlockSpec((r, c), lambda i: (0, 0))
    battn = lambda i: jnp.clip(i, 0, NBB - 1)
    call = _pc(
        _fused_b_body,
        grid=(NBB + NB + NC_STEPS,),
        in_specs=[
            pl.BlockSpec((BN, d0), lambda i: (battn(i) + NBA, 0)),
            pl.BlockSpec((20, BN, 2 * H), lambda i: (0, battn(i), 0)),
            pl.BlockSpec((5, BN, 256), lambda i: (0, battn(i), 0)),
            pl.BlockSpec((BN, H), lambda i: (jnp.clip(i - NBB, 0, NBA - 1),
                                             0)),
            pl.BlockSpec((BN, H), lambda i: (jnp.clip(i - NBB, 0, NBA - 1),
                                             0)),
            pl.BlockSpec((NBA, 1, H), lambda i: (0, 0, 0)),
            pl.BlockSpec((NBA, 1, H), lambda i: (0, 0, 0)),
            pl.BlockSpec((SB, N0),
                         lambda i: (jnp.clip(i - NBB - NB, 0,
                                             NC_STEPS - 1), 0)),
            full(d0, H), full(1, H), full(1, H), full(1, 2 * H),
            full(1, H), full(1, H), full(256, H), full(1, H),
            full(H, H), full(1, H), full(1, H),
            full(H, H), full(1, H), full(H, H), full(1, H),
        ],
        out_specs=pl.BlockSpec((1, 1), lambda i: (0, 0)),
        out_shape=jax.ShapeDtypeStruct((1, 1), jnp.float32),
        scratch_shapes=[
            pltpu.VMEM((NBB * BN, H), jnp.float32),
            pltpu.VMEM((NBB * BN, H), jnp.float32),
            pltpu.VMEM((N0, H), jnp.float32),
            pltpu.VMEM((1, H), jnp.float32),
            pltpu.VMEM((1, H), jnp.float32),
        ],
    )
    return call(feats0, g1r, g2r, e0a, e1a, ta0, ta1, pos,
                W0T, b0r, ar0, an0p, ar1, an1, W2T, b2r, WintT, bintr,
                att_inter, Wp1T, bp1r, Wp2T, bp2r)


def kernel(feats0, feats1, feats2, pos, nei0, nei1, W0, b0, W1, b1, W2, b2,
           att_i0, att_i1, Wint, bint, att_inter, Wp1, bp1, Wp2, bp2):
    h1 = _project_h1(feats1, W1, b1)

    nT0 = nei0.T
    nT1 = nei1.T
    z20 = jnp.zeros((20, 96), jnp.int32)
    z5 = jnp.zeros((5, 96), jnp.int32)
    n0A = jnp.concatenate([nT0[:, :4000], z20], axis=1).reshape(-1)
    n1A = jnp.concatenate([nT1[:, :4000], z5], axis=1).reshape(-1)
    n0B = jnp.concatenate([nT0[:, 4000:], z20[:, :24]], axis=1).reshape(-1)
    n1B = jnp.concatenate([nT1[:, 4000:], z5[:, :24]], axis=1).reshape(-1)

    g1A, g2A = _sc_gathers(n0A, h1, n1A, feats2)
    g1B, g2B = _sc_gathers(n0B, h1, n1B, feats2)

    W0T = W0.T
    b0r = b0.reshape(1, H)
    ar0 = att_i0[:, :H]
    an0p = jnp.concatenate([att_i0[:, H:], jnp.zeros((1, H), jnp.float32)],
                           axis=1)
    ar1 = att_i1[:, :H]
    an1 = att_i1[:, H:]
    W2T = W2.T
    b2r = b2.reshape(1, H)
    WintT = Wint.T
    bintr = bint.reshape(1, H)

    e0A, e1A, tA0, tA1 = _attn_a(
        feats0, g1A.reshape(20, 4096, 2 * H), g2A.reshape(5, 4096, 256),
        W0T, b0r, ar0, an0p, ar1, an1, W2T, b2r, WintT, bintr)

    out = _fused_b(
        feats0, g1B.reshape(20, 1024, 2 * H), g2B.reshape(5, 1024, 256),
        e0A, e1A, tA0, tA1, pos,
        W0T, b0r, ar0, an0p, ar1, an1, W2T, b2r, WintT, bintr,
        att_inter, Wp1.T, bp1.reshape(1, H), Wp2.T, bp2.reshape(1, H))
    return out[0, 0]
